# baseline (speedup 1.0000x reference)
import functools
import numpy as np
import jax
import jax.numpy as jnp
from jax.experimental import pallas as pl
from jax.experimental.pallas import tpu as pltpu
from jax.experimental.pallas import tpu_sc as plsc

ENC = 512
NODES = 63
LEVELS = [
    [62, 46, 38, 54, 34, 42, 50, 58, 32, 36, 40, 44, 48, 52, 56, 60],
    [30, 22, 18, 26, 16, 20, 24, 28],
    [14, 10, 8, 12],
    [6, 4],
    [2],
    [0],
]
BATCH = 16
NROWS = 512
NCORES = 2
NSUB = 16
NW = NCORES * NSUB
BPW = NROWS // NW

ORDER_FLAT = np.array([b * NODES + nd for level in LEVELS for nd in level
                       for b in range(BATCH)], dtype=np.int32)


def _sc_gather(tokens_flat, emb, order):
    mesh = plsc.VectorSubcoreMesh(core_axis_name="c", subcore_axis_name="s")

    @functools.partial(
        pl.kernel,
        mesh=mesh,
        out_type=jax.ShapeDtypeStruct((NROWS, ENC), jnp.float32),
        scratch_types=[
            pltpu.VMEM((BPW,), jnp.int32),
            pltpu.VMEM((BPW,), jnp.int32),
            pltpu.VMEM((BPW, ENC), jnp.float32),
            pltpu.SemaphoreType.DMA,
        ],
    )
    def gather_kernel(order_hbm, tok_hbm, emb_hbm, out_hbm,
                      ord_v, ids_v, rows_v, sem):
        wid = jax.lax.axis_index("s") * NCORES + jax.lax.axis_index("c")
        base = wid * BPW
        pltpu.sync_copy(order_hbm.at[pl.ds(base, BPW)], ord_v)
        pltpu.async_copy(tok_hbm.at[ord_v], ids_v, sem).wait()
        pltpu.async_copy(emb_hbm.at[ids_v], rows_v, sem).wait()
        pltpu.sync_copy(rows_v, out_hbm.at[pl.ds(base, BPW)])

    return gather_kernel(order, tokens_flat, emb)


_DN_T = (((1,), (1,)), ((), ()))


def _compute_body(x_ref, wih_ref, whh_hbm, bih_ref, bhh_ref, sw_hbm, sb_ref,
                  cw_ref, out_ref, whh_vmem, sw_vmem, sem_whh, sem_sw, B):
    whh_copy = pltpu.make_async_copy(whh_hbm, whh_vmem, sem_whh)
    sw_copy = pltpu.make_async_copy(sw_hbm, sw_vmem, sem_sw)
    whh_copy.start()
    sw_copy.start()

    b_ih = jnp.reshape(bih_ref[...], (1, 3 * ENC))
    b_hh = jnp.reshape(bhh_ref[...], (1, 3 * ENC))
    gi_all = jax.lax.dot_general(x_ref[...], wih_ref[...], _DN_T,
                                 preferred_element_type=jnp.float32) + b_ih
    sw_copy.wait()
    whh_copy.wait()
    sw = sw_vmem[...]
    sb = sb_ref[...]
    cw = cw_ref[...]
    c0 = jnp.dot(jnp.tanh(sb), cw, preferred_element_type=jnp.float32)

    out = jnp.zeros((B, ENC), dtype=jnp.float32)
    off = 0
    h_prev = None
    for s, level in enumerate(LEVELS):
        n = len(level) * B
        gi = gi_all[off:off + n]
        if s == 0:
            c = jnp.zeros((n, ENC), dtype=jnp.float32)
            gh = jnp.broadcast_to(b_hh, (n, 3 * ENC))
        else:
            h_child = h_prev[:n]
            t = jnp.tanh(jnp.dot(h_child, sw,
                                 preferred_element_type=jnp.float32) + sb)
            l = jnp.tanh(jnp.dot(t, cw, preferred_element_type=jnp.float32))
            k = 15.0 if s == len(LEVELS) - 1 else 1.0
            gate = 1.0 / (1.0 + k * jnp.exp(c0 - l))
            c = h_child * gate
            gh = jax.lax.dot_general(c, whh_vmem[...], _DN_T,
                                     preferred_element_type=jnp.float32) + b_hh
        i_r = gi[:, 0:ENC]
        i_z = gi[:, ENC:2 * ENC]
        i_n = gi[:, 2 * ENC:3 * ENC]
        h_r = gh[:, 0:ENC]
        h_z = gh[:, ENC:2 * ENC]
        h_n = gh[:, 2 * ENC:3 * ENC]
        r = jax.nn.sigmoid(i_r + h_r)
        z = jax.nn.sigmoid(i_z + h_z)
        nn_ = jnp.tanh(i_n + r * h_n)
        h = (1.0 - z) * nn_ + z * c
        for i in range(len(level)):
            out = jnp.maximum(out, h[i * B:(i + 1) * B])
        h_prev = h
        off += n
    out_ref[...] = jnp.maximum(out, 0.0)


@jax.jit
def _run(tokens, emb, W_ih, W_hh, b_ih, b_hh, sent_weight, sent_bias,
         context_weight):
    B = tokens.shape[0]
    order = jnp.asarray(ORDER_FLAT)
    x = _sc_gather(tokens.reshape(-1), emb, order)

    vm = pltpu.MemorySpace.VMEM
    hbm = pltpu.MemorySpace.HBM
    out = pl.pallas_call(
        functools.partial(_compute_body, B=B),
        in_specs=[
            pl.BlockSpec(memory_space=vm),
            pl.BlockSpec(memory_space=vm),
            pl.BlockSpec(memory_space=hbm),
            pl.BlockSpec(memory_space=vm),
            pl.BlockSpec(memory_space=vm),
            pl.BlockSpec(memory_space=hbm),
            pl.BlockSpec(memory_space=vm),
            pl.BlockSpec(memory_space=vm),
        ],
        scratch_shapes=[
            pltpu.VMEM((3 * ENC, ENC), jnp.float32),
            pltpu.VMEM((ENC, ENC), jnp.float32),
            pltpu.SemaphoreType.DMA,
            pltpu.SemaphoreType.DMA,
        ],
        out_shape=jax.ShapeDtypeStruct((B, ENC), jnp.float32),
    )(x, W_ih, W_hh, b_ih, b_hh, sent_weight, sent_bias, context_weight)
    return out


def kernel(tokens, bs, emb, W_ih, W_hh, b_ih, b_hh, sent_weight, sent_bias,
           context_weight):
    del bs
    return _run(tokens, emb, W_ih, W_hh, b_ih, b_hh, sent_weight,
                sent_bias, context_weight)


# speedup vs baseline: 2.2635x; 2.2635x over previous
import functools
import jax
import jax.numpy as jnp
from jax.experimental import pallas as pl
from jax.experimental.pallas import tpu as pltpu

ENC = 512
NODES = 63
BATCH = 16
LEVELS = [
    [62, 46, 38, 54, 34, 42, 50, 58, 32, 36, 40, 44, 48, 52, 56, 60],
    [30, 22, 18, 26, 16, 20, 24, 28],
    [14, 10, 8, 12],
    [6, 4],
    [2],
    [0],
]
ALL_NODES = [nd for level in LEVELS for nd in level]

_DN_T = (((1,), (1,)), ((), ()))


def _body(tok_ref, emb_hbm, wih_hbm, whh_hbm, bih_ref, bhh_ref, sw_hbm,
          sb_ref, cw_ref, out_ref, x_vmem, wih_vmem, whh_vmem, sw_vmem,
          sem_rows, sem_wih, sem_whh, sem_sw):
    B = BATCH
    wih_copy = pltpu.make_async_copy(wih_hbm, wih_vmem, sem_wih)
    whh_copy = pltpu.make_async_copy(whh_hbm, whh_vmem, sem_whh)
    sw_copy = pltpu.make_async_copy(sw_hbm, sw_vmem, sem_sw)
    wih_copy.start()
    whh_copy.start()
    sw_copy.start()

    copies = []
    r = 0
    for node in ALL_NODES:
        for b in range(B):
            tok = tok_ref[b, node]
            cp = pltpu.make_async_copy(emb_hbm.at[pl.ds(tok, 1)],
                                       x_vmem.at[pl.ds(r, 1)], sem_rows)
            cp.start()
            copies.append(cp)
            r += 1
    for cp in copies:
        cp.wait()
    wih_copy.wait()

    b_ih = jnp.reshape(bih_ref[...], (1, 3 * ENC))
    b_hh = jnp.reshape(bhh_ref[...], (1, 3 * ENC))
    gi_all = jax.lax.dot_general(x_vmem[...], wih_vmem[...], _DN_T,
                                 preferred_element_type=jnp.float32) + b_ih
    sw_copy.wait()
    whh_copy.wait()
    sw = sw_vmem[...]
    sb = sb_ref[...]
    cw = cw_ref[...]
    c0 = jnp.dot(jnp.tanh(sb), cw, preferred_element_type=jnp.float32)

    out = jnp.zeros((B, ENC), dtype=jnp.float32)
    off = 0
    h_prev = None
    for s, level in enumerate(LEVELS):
        n = len(level) * B
        gi = gi_all[off:off + n]
        if s == 0:
            c = jnp.zeros((n, ENC), dtype=jnp.float32)
            gh = jnp.broadcast_to(b_hh, (n, 3 * ENC))
        else:
            h_child = h_prev[:n]
            t = jnp.tanh(jnp.dot(h_child, sw,
                                 preferred_element_type=jnp.float32) + sb)
            l = jnp.tanh(jnp.dot(t, cw, preferred_element_type=jnp.float32))
            k = 15.0 if s == len(LEVELS) - 1 else 1.0
            gate = 1.0 / (1.0 + k * jnp.exp(c0 - l))
            c = h_child * gate
            gh = jax.lax.dot_general(c, whh_vmem[...], _DN_T,
                                     preferred_element_type=jnp.float32) + b_hh
        i_r = gi[:, 0:ENC]
        i_z = gi[:, ENC:2 * ENC]
        i_n = gi[:, 2 * ENC:3 * ENC]
        h_r = gh[:, 0:ENC]
        h_z = gh[:, ENC:2 * ENC]
        h_n = gh[:, 2 * ENC:3 * ENC]
        rr = jax.nn.sigmoid(i_r + h_r)
        z = jax.nn.sigmoid(i_z + h_z)
        nn_ = jnp.tanh(i_n + rr * h_n)
        h = (1.0 - z) * nn_ + z * c
        for i in range(len(level)):
            out = jnp.maximum(out, h[i * B:(i + 1) * B])
        h_prev = h
        off += n
    out_ref[...] = jnp.maximum(out, 0.0)


@jax.jit
def _run(tokens, emb, W_ih, W_hh, b_ih, b_hh, sent_weight, sent_bias,
         context_weight):
    vm = pltpu.MemorySpace.VMEM
    hbm = pltpu.MemorySpace.HBM
    smem = pltpu.MemorySpace.SMEM
    out = pl.pallas_call(
        _body,
        in_specs=[
            pl.BlockSpec(memory_space=smem),
            pl.BlockSpec(memory_space=hbm),
            pl.BlockSpec(memory_space=hbm),
            pl.BlockSpec(memory_space=hbm),
            pl.BlockSpec(memory_space=vm),
            pl.BlockSpec(memory_space=vm),
            pl.BlockSpec(memory_space=hbm),
            pl.BlockSpec(memory_space=vm),
            pl.BlockSpec(memory_space=vm),
        ],
        scratch_shapes=[
            pltpu.VMEM((32 * BATCH, ENC), jnp.float32),
            pltpu.VMEM((3 * ENC, ENC), jnp.float32),
            pltpu.VMEM((3 * ENC, ENC), jnp.float32),
            pltpu.VMEM((ENC, ENC), jnp.float32),
            pltpu.SemaphoreType.DMA,
            pltpu.SemaphoreType.DMA,
            pltpu.SemaphoreType.DMA,
            pltpu.SemaphoreType.DMA,
        ],
        out_shape=jax.ShapeDtypeStruct((BATCH, ENC), jnp.float32),
    )(tokens, emb, W_ih, W_hh, b_ih, b_hh, sent_weight, sent_bias,
      context_weight)
    return out


def kernel(tokens, bs, emb, W_ih, W_hh, b_ih, b_hh, sent_weight, sent_bias,
           context_weight):
    del bs
    return _run(tokens, emb, W_ih, W_hh, b_ih, b_hh, sent_weight,
                sent_bias, context_weight)
